# Initial kernel scaffold; baseline (speedup 1.0000x reference)
import jax
import jax.numpy as jnp
from jax.experimental import pallas as pl


def kernel(data, u_table, i_table):
    raise NotImplementedError("write your pallas kernel here")


# speedup vs baseline: 1.3079x; 1.3079x over previous
import functools

import jax
import jax.numpy as jnp
from jax import lax
from jax.experimental import pallas as pl
from jax.experimental.pallas import tpu as pltpu
from jax.experimental.pallas import tpu_sc as plsc

EMB = 64
NC, NS, LANES = 2, 16, 16
NW = NC * NS
CHUNK = 128
GROUPS = CHUNK // LANES


@functools.partial(jax.jit, static_argnames=("tot",))
def _run_sc(u_table, i_table, u_idx, i_idx, *, tot):
    npw = tot // NW
    nchunk = npw // CHUNK
    mesh = plsc.VectorSubcoreMesh(core_axis_name="c", subcore_axis_name="s")

    @functools.partial(
        pl.kernel,
        out_type=jax.ShapeDtypeStruct((tot,), jnp.float32),
        mesh=mesh,
        compiler_params=pltpu.CompilerParams(
            needs_layout_passes=False, use_tc_tiling_on_sc=False),
        scratch_types=[
            pltpu.VMEM((nchunk, CHUNK), jnp.int32),
            pltpu.VMEM((nchunk, CHUNK), jnp.int32),
            pltpu.VMEM((npw,), jnp.float32),
            pltpu.VMEM((CHUNK, EMB), jnp.float32),
            pltpu.VMEM((CHUNK, EMB), jnp.float32),
            pltpu.SemaphoreType.DMA,
            pltpu.SemaphoreType.DMA,
        ],
    )
    def sc_kernel(u_tab, i_tab, u_idx_hbm, i_idx_hbm, out_hbm,
                  u_idx_v, i_idx_v, out_v, u_rows, i_rows, su, si):
        wid = lax.axis_index("s") * NC + lax.axis_index("c")
        pltpu.sync_copy(u_idx_hbm.at[wid], u_idx_v)
        pltpu.sync_copy(i_idx_hbm.at[wid], i_idx_v)

        def chunk_body(k, carry):
            cu = pltpu.async_copy(u_tab.at[u_idx_v.at[k]], u_rows, su)
            ci = pltpu.async_copy(i_tab.at[i_idx_v.at[k]], i_rows, si)
            cu.wait()
            ci.wait()

            def group_body(g, c2):
                jvec = lax.iota(jnp.int32, LANES) + g * LANES
                acc = jnp.zeros((LANES,), jnp.float32)
                for d in range(EMB):
                    dcol = jnp.full((LANES,), d, jnp.int32)
                    uv = plsc.load_gather(u_rows, [jvec, dcol])
                    iv = plsc.load_gather(i_rows, [jvec, dcol])
                    acc = acc + uv * iv
                out_v[pl.ds(k * CHUNK + g * LANES, LANES)] = acc
                return c2

            lax.fori_loop(0, GROUPS, group_body, 0)
            return carry

        lax.fori_loop(0, nchunk, chunk_body, 0)
        pltpu.sync_copy(out_v, out_hbm.at[pl.ds(wid * npw, npw)])

    return sc_kernel(u_table, i_table, u_idx, i_idx)


def kernel(data, u_table, i_table):
    b, s, _ = data.shape
    tot = b * s
    flat = data.reshape(tot, 2).astype(jnp.int32)
    nchunk = tot // NW // CHUNK
    u_idx = flat[:, 0].reshape(NW, nchunk, CHUNK)
    i_idx = flat[:, 1].reshape(NW, nchunk, CHUNK)
    out = _run_sc(u_table, i_table, u_idx, i_idx, tot=tot)
    return out.reshape(b, s)
